# baseline (speedup 1.0000x reference)
import jax
import jax.numpy as jnp
from jax.experimental import pallas as pl
from jax.experimental.pallas import tpu as pltpu

_EPS = 1e-09
_W = 5000
_BLOCK_R = 256
_THIRD = 1.0 / 3.0


def _body(x_ref, o_ref):
    x = x_ref[...]
    r = x.shape[0]
    zero1 = jnp.zeros((r, 1), x.dtype)
    zero2 = jnp.zeros((r, 2), x.dtype)
    for _ in range(4):
        m = jnp.minimum(jnp.minimum(x[:, :-2], x[:, 1:-1]), x[:, 2:])
        pmin = jnp.min(m, axis=1, keepdims=True)
        pmax = jnp.max(m, axis=1, keepdims=True)
        inv = 1.0 / (pmax - pmin)
        m = (m - pmin) * inv + _EPS
        m = jnp.concatenate([m, zero2], axis=1)
        left = jnp.concatenate([zero1, m[:, :-1]], axis=1)
        right = jnp.concatenate([m[:, 1:], zero1], axis=1)
        x = (left + m + right) * _THIRD
    o_ref[...] = x.reshape(r, 1, 50, 100)


def kernel(x):
    x = x.reshape(-1, _W).astype(jnp.float32)
    n = x.shape[0]
    out = pl.pallas_call(
        _body,
        grid=(n // _BLOCK_R,),
        in_specs=[pl.BlockSpec((_BLOCK_R, _W), lambda i: (i, 0))],
        out_specs=pl.BlockSpec(
            (_BLOCK_R, 1, 50, 100), lambda i: (i, 0, 0, 0)
        ),
        out_shape=jax.ShapeDtypeStruct((n, 1, 50, 100), jnp.float32),
        compiler_params=pltpu.CompilerParams(
            dimension_semantics=("parallel",)
        ),
    )(x)
    return out


# speedup vs baseline: 1.3616x; 1.3616x over previous
import jax
import jax.numpy as jnp
from jax.experimental import pallas as pl
from jax.experimental.pallas import tpu as pltpu

_EPS = 1e-09
_W = 5000
_BLOCK_B = 256
_THIRD = 1.0 / 3.0


def _body(x_ref, o_ref):
    x = x_ref[...]
    b = x.shape[1]
    zero1 = jnp.zeros((1, b), x.dtype)
    zero2 = jnp.zeros((2, b), x.dtype)
    for _ in range(4):
        m = jnp.minimum(jnp.minimum(x[:-2], x[1:-1]), x[2:])
        pmin = jnp.min(m, axis=0, keepdims=True)
        pmax = jnp.max(m, axis=0, keepdims=True)
        inv = 1.0 / (pmax - pmin)
        m = (m - pmin) * inv + _EPS
        m = jnp.concatenate([m, zero2], axis=0)
        top = jnp.concatenate([zero1, m[:-1]], axis=0)
        bot = jnp.concatenate([m[1:], zero1], axis=0)
        x = (top + m + bot) * _THIRD
    o_ref[...] = x


def kernel(x):
    x = x.reshape(-1, _W).astype(jnp.float32)
    n = x.shape[0]
    xt = x.T
    out = pl.pallas_call(
        _body,
        grid=(n // _BLOCK_B,),
        in_specs=[pl.BlockSpec((_W, _BLOCK_B), lambda i: (0, i))],
        out_specs=pl.BlockSpec((_W, _BLOCK_B), lambda i: (0, i)),
        out_shape=jax.ShapeDtypeStruct((_W, n), jnp.float32),
        compiler_params=pltpu.CompilerParams(
            dimension_semantics=("parallel",)
        ),
    )(xt)
    return out.T.reshape(-1, 1, 50, 100)


# speedup vs baseline: 1.4855x; 1.0910x over previous
import jax
import jax.numpy as jnp
from jax.experimental import pallas as pl
from jax.experimental.pallas import tpu as pltpu

_EPS = 1e-09
_W = 5000
_BLOCK_B = 256
_THIRD = 1.0 / 3.0


def _body(x_ref, o_ref, m_ref, xb_ref):
    for it in range(4):
        src = x_ref if it == 0 else xb_ref
        dst = o_ref if it == 3 else xb_ref
        m_ref[1:4999] = jnp.minimum(
            jnp.minimum(src[0:4998], src[1:4999]), src[2:5000]
        )
        m = m_ref[1:4999]
        pmin = jnp.min(m, axis=0, keepdims=True)
        pmax = jnp.max(m, axis=0, keepdims=True)
        rng = pmax - pmin
        inv = 1.0 / rng
        bias = _EPS - pmin * inv
        q = pmin - _EPS * rng
        m_ref[0:1] = q
        m_ref[4999:5002] = jnp.broadcast_to(q, (3, q.shape[1]))
        dst[...] = (
            m_ref[0:5000] + m_ref[1:5001] + m_ref[2:5002]
        ) * (inv * _THIRD) + bias


def kernel(x):
    x = x.reshape(-1, _W).astype(jnp.float32)
    n = x.shape[0]
    xt = x.T
    out = pl.pallas_call(
        _body,
        grid=(n // _BLOCK_B,),
        in_specs=[pl.BlockSpec((_W, _BLOCK_B), lambda i: (0, i))],
        out_specs=pl.BlockSpec((_W, _BLOCK_B), lambda i: (0, i)),
        out_shape=jax.ShapeDtypeStruct((_W, n), jnp.float32),
        scratch_shapes=[
            pltpu.VMEM((_W + 2, _BLOCK_B), jnp.float32),
            pltpu.VMEM((_W, _BLOCK_B), jnp.float32),
        ],
        compiler_params=pltpu.CompilerParams(
            dimension_semantics=("parallel",)
        ),
    )(xt)
    return out.T.reshape(-1, 1, 50, 100)


# speedup vs baseline: 1.5116x; 1.0176x over previous
import jax
import jax.numpy as jnp
from jax.experimental import pallas as pl
from jax.experimental.pallas import tpu as pltpu

_EPS = 1e-09
_W = 5000
_BLOCK_B = 256
_THIRD = 1.0 / 3.0


def _body(x_ref, o_ref, m_ref, xb_ref):
    for it in range(4):
        src = x_ref if it == 0 else xb_ref
        dst = o_ref if it == 3 else xb_ref
        m_ref[1:4999] = jnp.minimum(
            jnp.minimum(src[0:4998], src[1:4999]), src[2:5000]
        )
        m = m_ref[1:4999]
        pmin = jnp.min(m, axis=0, keepdims=True)
        pmax = jnp.max(m, axis=0, keepdims=True)
        rng = pmax - pmin
        inv = 1.0 / rng
        q = pmin - _EPS * rng
        m_ref[0:1] = q
        m_ref[4999:5002] = jnp.broadcast_to(q, (3, q.shape[1]))
        s3 = (m_ref[0:5000] + m_ref[1:5001] + m_ref[2:5002]) * (inv * _THIRD)
        if it == 3:
            dst[...] = s3 + (_EPS - pmin * inv)
        else:
            dst[...] = s3


def kernel(x):
    x = x.reshape(-1, _W).astype(jnp.float32)
    n = x.shape[0]
    xt = x.T
    out = pl.pallas_call(
        _body,
        grid=(n // _BLOCK_B,),
        in_specs=[pl.BlockSpec((_W, _BLOCK_B), lambda i: (0, i))],
        out_specs=pl.BlockSpec((_W, _BLOCK_B), lambda i: (0, i)),
        out_shape=jax.ShapeDtypeStruct((_W, n), jnp.float32),
        scratch_shapes=[
            pltpu.VMEM((_W + 2, _BLOCK_B), jnp.float32),
            pltpu.VMEM((_W, _BLOCK_B), jnp.float32),
        ],
        compiler_params=pltpu.CompilerParams(
            dimension_semantics=("parallel",)
        ),
    )(xt)
    return out.T.reshape(-1, 1, 50, 100)
